# baseline (speedup 1.0000x reference)
import functools

import jax
import jax.numpy as jnp
from jax import lax
from jax.experimental import pallas as pl
from jax.experimental.pallas import tpu as pltpu
from jax.experimental.pallas import tpu_sc as plsc

_N_TYPES = 100000
_D = 64
_B = 16384

_NC = 2
_NS = 16
_NW = _NC * _NS
_BPW = _B // _NW
_G = 16
_NG = _BPW // _G

_TBLK = 1024
_TGRID = -(-_N_TYPES // _TBLK)

_mesh = plsc.VectorSubcoreMesh(core_axis_name="c", subcore_axis_name="s")


@functools.partial(
    pl.kernel,
    mesh=_mesh,
    out_type=jax.ShapeDtypeStruct((_B, _D), jnp.float32),
    compiler_params=pltpu.CompilerParams(use_tc_tiling_on_sc=True),
    scratch_types=[
        pltpu.VMEM((_BPW,), jnp.int32),
        pltpu.SemaphoreType.DMA,
    ],
)
def _gather(table_hbm, idx_hbm, out_hbm, idx_v, sem0):
    wid = lax.axis_index("s") * _NC + lax.axis_index("c")
    base = wid * _BPW
    pltpu.sync_copy(idx_hbm.at[pl.ds(base, _BPW)], idx_v)

    def body(g, _):
        vec = idx_v[pl.ds(g * _G, _G)]
        for l in range(_G):
            pltpu.async_copy(
                table_hbm.at[pl.ds(vec[l], 1)],
                out_hbm.at[pl.ds(base + g * _G + l, 1)],
                sem0,
            )
        return 0

    lax.fori_loop(0, _NG, body, 0)
    pltpu.make_async_copy(
        table_hbm.at[pl.ds(0, _BPW)],
        out_hbm.at[pl.ds(base, _BPW)],
        sem0,
    ).wait()


def kernel(idx, table):
    return _gather(table, idx.astype(jnp.int32))


# speedup vs baseline: 4.7668x; 4.7668x over previous
import functools

import jax
import jax.numpy as jnp
from jax import lax
from jax.experimental import pallas as pl
from jax.experimental.pallas import tpu as pltpu
from jax.experimental.pallas import tpu_sc as plsc

_N_TYPES = 100000
_D = 64
_B = 16384

_NC = 2
_NS = 16
_NW = _NC * _NS
_BPW = _B // _NW
_G = 16
_NG = _BPW // _G

_TBLK = 1024
_TGRID = -(-_N_TYPES // _TBLK)

_mesh = plsc.VectorSubcoreMesh(core_axis_name="c", subcore_axis_name="s")


@functools.partial(
    pl.kernel,
    mesh=_mesh,
    out_type=jax.ShapeDtypeStruct((_B, _D), jnp.float32),
    compiler_params=pltpu.CompilerParams(use_tc_tiling_on_sc=True),
    scratch_types=[
        pltpu.VMEM((_BPW,), jnp.int32),
        pltpu.VMEM((_BPW, _D), jnp.float32),
        pltpu.SemaphoreType.DMA,
    ],
)
def _gather(table_hbm, idx_hbm, out_hbm, idx_v, rows_v, sem0):
    wid = lax.axis_index("s") * _NC + lax.axis_index("c")
    base = wid * _BPW
    pltpu.sync_copy(idx_hbm.at[pl.ds(base, _BPW)], idx_v)

    def body(g, _):
        vec = idx_v[pl.ds(g * _G, _G)]
        for l in range(_G):
            pltpu.async_copy(
                table_hbm.at[pl.ds(vec[l], 1)],
                rows_v.at[pl.ds(g * _G + l, 1)],
                sem0,
            )
        return 0

    lax.fori_loop(0, _NG, body, 0)
    pltpu.make_async_copy(table_hbm.at[pl.ds(0, _BPW)], rows_v, sem0).wait()
    pltpu.sync_copy(rows_v, out_hbm.at[pl.ds(base, _BPW)])


def kernel(idx, table):
    return _gather(table, idx.astype(jnp.int32))
